# Initial kernel scaffold; baseline (speedup 1.0000x reference)
import jax
import jax.numpy as jnp
from jax.experimental import pallas as pl


def kernel(token_ids, tok_table, pos_table):
    raise NotImplementedError("write your pallas kernel here")


# speedup vs baseline: 1.5634x; 1.5634x over previous
import functools

import jax
import jax.numpy as jnp
from jax import lax
from jax.experimental import pallas as pl
from jax.experimental.pallas import tpu as pltpu
from jax.experimental.pallas import tpu_sc as plsc

_B, _S, _D = 4, 4096, 2048
_N = _B * _S
_NC, _NS = 2, 16
_NW = _NC * _NS
_RPW = _N // _NW
_C = 8
_NCH = _RPW // _C
_VPC = _C * _D // 16

_mesh = plsc.VectorSubcoreMesh(core_axis_name="c", subcore_axis_name="s")


@functools.partial(
    pl.kernel,
    out_type=jax.ShapeDtypeStruct((_N, _D), jnp.float32),
    mesh=_mesh,
    scratch_types=[
        pltpu.VMEM((_RPW,), jnp.int32),
        pltpu.VMEM((2, _C, _D), jnp.float32),
        pltpu.VMEM((2, _C, _D), jnp.float32),
        pltpu.VMEM((2, _C, _D), jnp.float32),
        pltpu.SemaphoreType.DMA,
        pltpu.SemaphoreType.DMA,
        pltpu.SemaphoreType.DMA,
        pltpu.SemaphoreType.DMA,
        pltpu.SemaphoreType.DMA,
        pltpu.SemaphoreType.DMA,
    ],
)
def _embed_sc(ids_hbm, tok_hbm, pos_hbm, out_hbm, idx_v, pos_v, tok_v, res_v,
              sp0, sp1, st0, st1, so0, so1):
    wid = lax.axis_index("s") * _NC + lax.axis_index("c")
    base = wid * _RPW
    pos0 = base % _S
    pltpu.sync_copy(ids_hbm.at[pl.ds(base, _RPW)], idx_v)

    psems = (sp0, sp1)
    tsems = (st0, st1)
    osems = (so0, so1)

    def issue_in(c, b):
        r0 = c * _C
        pltpu.async_copy(pos_hbm.at[pl.ds(pos0 + r0, _C)], pos_v.at[b], psems[b])
        pltpu.async_copy(tok_hbm.at[idx_v.at[pl.ds(r0, _C)]], tok_v.at[b], tsems[b])

    issue_in(0, 0)
    issue_in(1, 1)

    @pl.loop(0, _NCH, step=2)
    def _chunks(j):
        for b in (0, 1):
            c = j + b
            pltpu.make_async_copy(pos_hbm.at[pl.ds(0, _C)], pos_v.at[b], psems[b]).wait()
            pltpu.make_async_copy(pos_hbm.at[pl.ds(0, _C)], tok_v.at[b], tsems[b]).wait()
            @pl.when(c >= 2)
            def _():
                pltpu.make_async_copy(pos_hbm.at[pl.ds(0, _C)], res_v.at[b], osems[b]).wait()

            pb, tb, rb = pos_v.at[b], tok_v.at[b], res_v.at[b]

            @plsc.parallel_loop(0, _VPC, unroll=8)
            def _add(t):
                i = lax.shift_right_logical(t, 7)
                k = pl.multiple_of(lax.shift_left(lax.bitwise_and(t, 127), 4), 16)
                rb[i, pl.ds(k, 16)] = tb[i, pl.ds(k, 16)] + pb[i, pl.ds(k, 16)]

            pltpu.async_copy(rb, out_hbm.at[pl.ds(base + c * _C, _C)], osems[b])

            @pl.when(c + 2 < _NCH)
            def _():
                issue_in(c + 2, b)

    for b in (0, 1):
        pltpu.make_async_copy(pos_hbm.at[pl.ds(0, _C)], res_v.at[b], osems[b]).wait()


def kernel(token_ids, tok_table, pos_table):
    ids = token_ids.reshape(_N).astype(jnp.int32)
    out = _embed_sc(ids, tok_table, pos_table)
    return out.reshape(_B, _S, _D)


# speedup vs baseline: 1.9751x; 1.2634x over previous
import functools

import jax
import jax.numpy as jnp
from jax import lax
from jax.experimental import pallas as pl
from jax.experimental.pallas import tpu as pltpu
from jax.experimental.pallas import tpu_sc as plsc

_B, _S, _D = 4, 4096, 2048
_N = _B * _S
_NC, _NS = 2, 16
_NW = _NC * _NS
_SPW = _S // _NW
_CS = 8
_NCS = _SPW // _CS
_VPU = _CS * _D // 16

_mesh = plsc.VectorSubcoreMesh(core_axis_name="c", subcore_axis_name="s")


@functools.partial(
    pl.kernel,
    out_type=jax.ShapeDtypeStruct((_N, _D), jnp.float32),
    mesh=_mesh,
    scratch_types=[
        pltpu.VMEM((_B, _SPW), jnp.int32),
        pltpu.VMEM((2, _CS, _D), jnp.float32),
        pltpu.VMEM((_B, _CS, _D), jnp.float32),
        pltpu.SemaphoreType.DMA,
        pltpu.SemaphoreType.DMA,
        pltpu.SemaphoreType.DMA,
        pltpu.SemaphoreType.DMA,
        pltpu.SemaphoreType.DMA,
        pltpu.SemaphoreType.DMA,
        pltpu.SemaphoreType.DMA,
        pltpu.SemaphoreType.DMA,
        pltpu.SemaphoreType.DMA,
    ],
)
def _embed_sc(ids_hbm, tok_hbm, pos_hbm, out_hbm, idx_v, pos_v, res_v,
              sem_pos, sg0, sg1, sg2, sg3, so0, so1, so2, so3):
    wid = lax.axis_index("s") * _NC + lax.axis_index("c")
    s0 = wid * _SPW
    gsems = (sg0, sg1, sg2, sg3)
    osems = (so0, so1, so2, so3)

    for b in range(_B):
        pltpu.sync_copy(ids_hbm.at[b, pl.ds(s0, _SPW)], idx_v.at[b])

    def issue_pos(cs, p):
        pltpu.async_copy(pos_hbm.at[pl.ds(s0 + cs * _CS, _CS)], pos_v.at[p],
                         sem_pos)

    def issue_gather(cs, b):
        pltpu.async_copy(tok_hbm.at[idx_v.at[b, pl.ds(cs * _CS, _CS)]],
                         res_v.at[b], gsems[b])

    def wait_gather(b):
        pltpu.make_async_copy(pos_hbm.at[pl.ds(0, _CS)], res_v.at[b],
                              gsems[b]).wait()

    def wait_out(b):
        pltpu.make_async_copy(pos_hbm.at[pl.ds(0, _CS)], res_v.at[b],
                              osems[b]).wait()

    issue_pos(0, 0)
    for b in range(_B):
        issue_gather(0, b)

    @pl.loop(0, _NCS, step=2)
    def _chunks(j):
        for p in (0, 1):
            cs = j + p
            pltpu.make_async_copy(pos_hbm.at[pl.ds(0, _CS)], pos_v.at[p],
                                  sem_pos).wait()

            @pl.when(cs + 1 < _NCS)
            def _():
                issue_pos(cs + 1, 1 - p)

            pos_p = pos_v.at[p]
            for b in range(_B):
                wait_gather(b)
                res_b = res_v.at[b]

                @plsc.parallel_loop(0, _VPU, unroll=8)
                def _add(t):
                    i = lax.shift_right_logical(t, 7)
                    k = pl.multiple_of(
                        lax.shift_left(lax.bitwise_and(t, 127), 4), 16)
                    plsc.addupdate(res_b.at[i, pl.ds(k, 16)],
                                   pos_p[i, pl.ds(k, 16)])

                pltpu.async_copy(
                    res_b,
                    out_hbm.at[pl.ds(b * _S + s0 + cs * _CS, _CS)],
                    osems[b])

                if b > 0:
                    @pl.when(cs + 1 < _NCS)
                    def _():
                        wait_out(b - 1)
                        issue_gather(cs + 1, b - 1)

            @pl.when(cs + 1 < _NCS)
            def _():
                wait_out(_B - 1)
                issue_gather(cs + 1, _B - 1)

    for b in range(_B):
        wait_out(b)


def kernel(token_ids, tok_table, pos_table):
    ids = token_ids.astype(jnp.int32)
    out = _embed_sc(ids, tok_table, pos_table)
    return out.reshape(_B, _S, _D)


# speedup vs baseline: 2.0369x; 1.0313x over previous
import functools

import jax
import jax.numpy as jnp
from jax import lax
from jax.experimental import pallas as pl
from jax.experimental.pallas import tpu as pltpu
from jax.experimental.pallas import tpu_sc as plsc

_B, _S, _D = 4, 4096, 2048
_N = _B * _S
_NC, _NS = 2, 16
_NW = _NC * _NS
_SPW = _S // _NW
_CS = 4
_NCS = _SPW // _CS
_NSL = 2 * _B
_VPU = _CS * _D // 16

_mesh = plsc.VectorSubcoreMesh(core_axis_name="c", subcore_axis_name="s")


@functools.partial(
    pl.kernel,
    out_type=jax.ShapeDtypeStruct((_N, _D), jnp.float32),
    mesh=_mesh,
    scratch_types=[
        pltpu.VMEM((_B, _SPW), jnp.int32),
        pltpu.VMEM((2, _CS, _D), jnp.float32),
        pltpu.VMEM((_NSL, _CS, _D), jnp.float32),
        pltpu.SemaphoreType.DMA,
    ] + [pltpu.SemaphoreType.DMA] * _NSL
      + [pltpu.SemaphoreType.DMA] * _NSL,
)
def _embed_sc(ids_hbm, tok_hbm, pos_hbm, out_hbm, idx_v, pos_v, res_v,
              sem_pos, *sems):
    gsems = sems[:_NSL]
    osems = sems[_NSL:]
    wid = lax.axis_index("s") * _NC + lax.axis_index("c")
    s0 = wid * _SPW

    def issue_pos(cs, p):
        pltpu.async_copy(pos_hbm.at[pl.ds(s0 + cs * _CS, _CS)], pos_v.at[p],
                         sem_pos)

    def issue_gather(cs, b, sl):
        pltpu.async_copy(tok_hbm.at[idx_v.at[b, pl.ds(cs * _CS, _CS)]],
                         res_v.at[sl], gsems[sl])

    def wait_gather(sl):
        pltpu.make_async_copy(pos_hbm.at[pl.ds(0, _CS)], res_v.at[sl],
                              gsems[sl]).wait()

    def wait_out(sl):
        pltpu.make_async_copy(pos_hbm.at[pl.ds(0, _CS)], res_v.at[sl],
                              osems[sl]).wait()

    for b in range(_B):
        pltpu.async_copy(ids_hbm.at[b, pl.ds(s0, _SPW)], idx_v.at[b],
                         osems[b])
    issue_pos(0, 0)
    for b in range(_B):
        pltpu.make_async_copy(ids_hbm.at[0, pl.ds(0, _SPW)], idx_v.at[b],
                              osems[b]).wait()
    for b in range(_B):
        issue_gather(0, b, b)

    @pl.loop(0, _NCS, step=2)
    def _chunks(j):
        for p in (0, 1):
            cs = j + p
            pltpu.make_async_copy(pos_hbm.at[pl.ds(0, _CS)], pos_v.at[p],
                                  sem_pos).wait()

            @pl.when(cs + 1 < _NCS)
            def _():
                issue_pos(cs + 1, 1 - p)

            pos_p = pos_v.at[p]
            for b in range(_B):
                sl = p * _B + b
                osl = (1 - p) * _B + b
                wait_gather(sl)
                res_b = res_v.at[sl]

                @plsc.parallel_loop(0, _VPU, unroll=8)
                def _add(t):
                    i = lax.shift_right_logical(t, 7)
                    k = pl.multiple_of(
                        lax.shift_left(lax.bitwise_and(t, 127), 4), 16)
                    plsc.addupdate(res_b.at[i, pl.ds(k, 16)],
                                   pos_p[i, pl.ds(k, 16)])

                pltpu.async_copy(
                    res_b,
                    out_hbm.at[pl.ds(b * _S + s0 + cs * _CS, _CS)],
                    osems[sl])

                @pl.when(cs >= 1)
                def _():
                    wait_out(osl)

                @pl.when(cs + 1 < _NCS)
                def _():
                    issue_gather(cs + 1, b, osl)

    for b in range(_B):
        wait_out(((_NCS - 1) % 2) * _B + b)


def kernel(token_ids, tok_table, pos_table):
    ids = token_ids.astype(jnp.int32)
    out = _embed_sc(ids, tok_table, pos_table)
    return out.reshape(_B, _S, _D)
